# baseline (speedup 1.0000x reference)
import functools

import jax
import jax.numpy as jnp
from jax import lax
from jax.experimental import pallas as pl
from jax.experimental.pallas import tpu as pltpu
from jax.experimental.pallas import tpu_sc as plsc

_N = 10000
_E = 320000
_DIN = 128
_DHID = 128
_NCLS = 40
_DP = 48

_NC = 2
_NS = 16
_NW = _NC * _NS
_EPW = _E // _NW
_CHUNK2 = 125
_WSTEP = 624
_WROWS = 640

_BN = 10000


def _make_segsum(D, chunk, halves):
  nchunks = _EPW // chunk
  nh = nchunks // halves
  mesh = plsc.VectorSubcoreMesh(core_axis_name="c", subcore_axis_name="s")

  @functools.partial(
      pl.kernel,
      out_type=jax.ShapeDtypeStruct((_NC * _N, D), jnp.float32),
      mesh=mesh,
      compiler_params=pltpu.CompilerParams(use_tc_tiling_on_sc=False,
                                           skip_device_barrier=True),
      scratch_types=[
          pltpu.VMEM((nh, chunk), jnp.int32),
          pltpu.VMEM((nh, chunk), jnp.int32),
          pltpu.VMEM((chunk, D), jnp.float32),
          pltpu.VMEM((chunk, D), jnp.float32),
          pltpu.VMEM_SHARED((_N, D), jnp.float32),
          pltpu.SemaphoreType.DMA,
          pltpu.SemaphoreType.DMA,
      ],
  )
  def segsum(g_hbm, src_hbm, dst_hbm, out_hbm,
             src_v, dst_v, rows0, rows1, acc_sh, gs0, gs1):
    cid = lax.axis_index("c")
    sid = lax.axis_index("s")
    wid = cid * _NS + sid
    z16 = jnp.zeros((16,), jnp.float32)

    def zrow(r, c):
      for j in range(D // 16):
        rows0[r, pl.ds(j * 16, 16)] = z16
      return c

    lax.fori_loop(0, chunk, zrow, 0)
    for c in range(5):
      pltpu.sync_copy(rows0, acc_sh.at[pl.ds(sid * 625 + c * 125, 125)])
    plsc.subcore_barrier()

    for h in range(halves):
      pltpu.sync_copy(src_hbm.at[pl.ds(wid * nchunks + h * nh, nh)], src_v)
      pltpu.sync_copy(dst_hbm.at[pl.ds(wid * nchunks + h * nh, nh)], dst_v)
      pltpu.async_copy(g_hbm.at[src_v.at[0]], rows0, gs0)

      def epair(k, carry):
        i = 2 * k
        i2 = jnp.minimum(i + 2, nh - 1)
        pltpu.async_copy(g_hbm.at[src_v.at[i + 1]], rows1, gs1)
        pltpu.make_async_copy(g_hbm.at[src_v.at[i]], rows0, gs0).wait()
        pltpu.sync_copy(rows0, acc_sh.at[dst_v.at[i]], add=True)
        pltpu.async_copy(g_hbm.at[src_v.at[i2]], rows0, gs0)
        pltpu.make_async_copy(g_hbm.at[src_v.at[i + 1]], rows1, gs1).wait()
        pltpu.sync_copy(rows1, acc_sh.at[dst_v.at[i + 1]], add=True)
        return carry

      lax.fori_loop(0, nh // 2, epair, 0)
      pltpu.make_async_copy(g_hbm.at[src_v.at[nh - 1]], rows0, gs0).wait()

    plsc.subcore_barrier()
    pltpu.sync_copy(acc_sh.at[pl.ds(sid * _WSTEP, _WROWS)],
                    out_hbm.at[pl.ds(cid * _N + sid * _WSTEP, _WROWS)])

  return segsum


_segsum_hid = _make_segsum(_DHID, _CHUNK2, 2)
_segsum_cls = _make_segsum(_DP, _CHUNK2, 1)

_deg_mesh = plsc.VectorSubcoreMesh(core_axis_name="c", subcore_axis_name="s")


@functools.partial(
    pl.kernel,
    out_type=jax.ShapeDtypeStruct((_NC * _N, 16), jnp.float32),
    mesh=_deg_mesh,
    compiler_params=pltpu.CompilerParams(use_tc_tiling_on_sc=False,
                                         skip_device_barrier=True),
    scratch_types=[
        pltpu.VMEM((_EPW // _CHUNK2, _CHUNK2), jnp.int32),
        pltpu.VMEM((_CHUNK2, 16), jnp.float32),
        pltpu.VMEM_SHARED((_N, 16), jnp.float32),
        pltpu.SemaphoreType.DMA,
    ],
)
def _sc_deg(dst_hbm, ones_hbm, z_hbm, out_hbm, dst_v, ones_v, acc_sh, sem):
  cid = lax.axis_index("c")
  sid = lax.axis_index("s")
  wid = cid * _NS + sid
  pltpu.sync_copy(z_hbm, acc_sh.at[pl.ds(sid * _WSTEP, _WROWS)])
  pltpu.sync_copy(dst_hbm.at[pl.ds(wid * (_EPW // _CHUNK2), _EPW // _CHUNK2)], dst_v)
  pltpu.sync_copy(ones_hbm, ones_v)
  plsc.subcore_barrier()

  def egroup(g, carry):
    def fire(j, c):
      pltpu.async_copy(ones_v, acc_sh.at[dst_v.at[g * 40 + j]], sem, add=True)
      return c

    lax.fori_loop(0, 40, fire, 0)

    def drain(j, c):
      pltpu.make_async_copy(ones_v, acc_sh.at[dst_v.at[0]], sem).wait()
      return c

    lax.fori_loop(0, 40, drain, 0)
    return carry

  lax.fori_loop(0, (_EPW // _CHUNK2) // 40, egroup, 0)
  plsc.subcore_barrier()
  pltpu.sync_copy(acc_sh.at[pl.ds(sid * _WSTEP, _WROWS)],
                  out_hbm.at[pl.ds(cid * _N + sid * _WSTEP, _WROWS)])


def _tc_prep(x, W1, degp):
  def body(x_ref, w_ref, d_ref, g_ref, dinv_ref):
    deg = d_ref[0, :, 0:1] + d_ref[1, :, 0:1] + 1.0
    dinv = lax.rsqrt(deg)
    g_ref[...] = jnp.dot(x_ref[...], w_ref[...],
                         preferred_element_type=jnp.float32) * dinv
    dinv_ref[...] = dinv

  return pl.pallas_call(
      body,
      grid=(_N // _BN,),
      in_specs=[
          pl.BlockSpec((_BN, _DIN), lambda i: (i, 0)),
          pl.BlockSpec((_DIN, _DHID), lambda i: (0, 0)),
          pl.BlockSpec((2, _BN, 16), lambda i: (0, i, 0)),
      ],
      out_specs=[
          pl.BlockSpec((_BN, _DHID), lambda i: (i, 0)),
          pl.BlockSpec((_BN, 1), lambda i: (i, 0)),
      ],
      out_shape=[
          jax.ShapeDtypeStruct((_N, _DHID), jnp.float32),
          jax.ShapeDtypeStruct((_N, 1), jnp.float32),
      ],
  )(x, W1, degp)


def _tc_mid(s1, g1, dinv, b1, W2p):
  def body(s_ref, g_ref, dv_ref, b_ref, w_ref, g2_ref):
    dinv = dv_ref[...]
    o1 = (s_ref[0] + s_ref[1] + g_ref[...]) * dinv + b_ref[...]
    o1 = jnp.maximum(o1, 0.0)
    g2_ref[...] = jnp.dot(o1, w_ref[...],
                          preferred_element_type=jnp.float32) * dinv

  return pl.pallas_call(
      body,
      grid=(_N // _BN,),
      in_specs=[
          pl.BlockSpec((2, _BN, _DHID), lambda i: (0, i, 0)),
          pl.BlockSpec((_BN, _DHID), lambda i: (i, 0)),
          pl.BlockSpec((_BN, 1), lambda i: (i, 0)),
          pl.BlockSpec((1, _DHID), lambda i: (0, 0)),
          pl.BlockSpec((_DHID, _DP), lambda i: (0, 0)),
      ],
      out_specs=pl.BlockSpec((_BN, _DP), lambda i: (i, 0)),
      out_shape=jax.ShapeDtypeStruct((_N, _DP), jnp.float32),
  )(s1, g1, dinv, b1, W2p)


def _tc_final(s2, g2, dinv, b2p):
  def body(s_ref, g_ref, dv_ref, b_ref, o_ref):
    dinv = dv_ref[...]
    t = (s_ref[0] + s_ref[1] + g_ref[...]) * dinv + b_ref[...]
    t = jnp.maximum(t, 0.0)
    mask = lax.broadcasted_iota(jnp.int32, (_BN, _DP), 1) < _NCLS
    t = jnp.where(mask, t, -1e30)
    m = jnp.max(t, axis=1, keepdims=True)
    e = jnp.exp(t - m)
    s = jnp.sum(e, axis=1, keepdims=True)
    o_ref[...] = (t - m - jnp.log(s))[:, :_NCLS]

  return pl.pallas_call(
      body,
      grid=(_N // _BN,),
      in_specs=[
          pl.BlockSpec((2, _BN, _DP), lambda i: (0, i, 0)),
          pl.BlockSpec((_BN, _DP), lambda i: (i, 0)),
          pl.BlockSpec((_BN, 1), lambda i: (i, 0)),
          pl.BlockSpec((1, _DP), lambda i: (0, 0)),
      ],
      out_specs=pl.BlockSpec((_BN, _NCLS), lambda i: (i, 0)),
      out_shape=jax.ShapeDtypeStruct((_N, _NCLS), jnp.float32),
  )(s2, g2, dinv, b2p)


def kernel(x, edge_index, W1, b1, W2, b2):
  src2 = edge_index[0].astype(jnp.int32).reshape(_E // _CHUNK2, _CHUNK2)
  dst2 = edge_index[1].astype(jnp.int32).reshape(_E // _CHUNK2, _CHUNK2)

  ones16 = jnp.ones((_CHUNK2, 16), jnp.float32)
  z16 = jnp.zeros((_WROWS, 16), jnp.float32)

  degp = _sc_deg(dst2, ones16, z16).reshape(2, _N, 16)
  g1, dinv = _tc_prep(x, W1, degp)
  s1 = _segsum_hid(g1, src2, dst2).reshape(2, _N, _DHID)
  g2 = _tc_mid(s1, g1, dinv, b1.reshape(1, _DHID),
               jnp.zeros((_DHID, _DP), jnp.float32).at[:, :_NCLS].set(W2))
  s2 = _segsum_cls(g2, src2, dst2).reshape(2, _N, _DP)
  return _tc_final(s2, g2, dinv,
                   jnp.zeros((1, _DP), jnp.float32).at[0, :_NCLS].set(b2))


# speedup vs baseline: 1.0188x; 1.0188x over previous
import functools

import jax
import jax.numpy as jnp
from jax import lax
from jax.experimental import pallas as pl
from jax.experimental.pallas import tpu as pltpu
from jax.experimental.pallas import tpu_sc as plsc

_N = 10000
_E = 320000
_DIN = 128
_DHID = 128
_NCLS = 40
_DP = 48

_NC = 2
_NS = 16
_NW = _NC * _NS
_EPW = _E // _NW
_CHUNK2 = 125
_WSTEP = 624
_WROWS = 640

_BN = 5000


def _make_segsum(D, chunk, halves):
  nchunks = _EPW // chunk
  nh = nchunks // halves
  mesh = plsc.VectorSubcoreMesh(core_axis_name="c", subcore_axis_name="s")

  @functools.partial(
      pl.kernel,
      out_type=jax.ShapeDtypeStruct((_NC * _N, D), jnp.float32),
      mesh=mesh,
      compiler_params=pltpu.CompilerParams(use_tc_tiling_on_sc=False,
                                           skip_device_barrier=True),
      scratch_types=[
          pltpu.VMEM((nh, chunk), jnp.int32),
          pltpu.VMEM((nh, chunk), jnp.int32),
          pltpu.VMEM((chunk, D), jnp.float32),
          pltpu.VMEM((chunk, D), jnp.float32),
          pltpu.VMEM_SHARED((_N, D), jnp.float32),
          pltpu.SemaphoreType.DMA,
          pltpu.SemaphoreType.DMA,
      ],
  )
  def segsum(g_hbm, src_hbm, dst_hbm, out_hbm,
             src_v, dst_v, rows0, rows1, acc_sh, gs0, gs1):
    cid = lax.axis_index("c")
    sid = lax.axis_index("s")
    wid = cid * _NS + sid
    z16 = jnp.zeros((16,), jnp.float32)

    def zrow(r, c):
      for j in range(D // 16):
        rows0[r, pl.ds(j * 16, 16)] = z16
      return c

    lax.fori_loop(0, chunk, zrow, 0)
    for c in range(5):
      pltpu.sync_copy(rows0, acc_sh.at[pl.ds(sid * 625 + c * 125, 125)])
    plsc.subcore_barrier()

    for h in range(halves):
      pltpu.sync_copy(src_hbm.at[pl.ds(wid * nchunks + h * nh, nh)], src_v)
      pltpu.sync_copy(dst_hbm.at[pl.ds(wid * nchunks + h * nh, nh)], dst_v)
      pltpu.async_copy(g_hbm.at[src_v.at[0]], rows0, gs0)

      def epair(k, carry):
        i = 2 * k
        i2 = jnp.minimum(i + 2, nh - 1)
        pltpu.async_copy(g_hbm.at[src_v.at[i + 1]], rows1, gs1)
        pltpu.make_async_copy(g_hbm.at[src_v.at[i]], rows0, gs0).wait()
        pltpu.sync_copy(rows0, acc_sh.at[dst_v.at[i]], add=True)
        pltpu.async_copy(g_hbm.at[src_v.at[i2]], rows0, gs0)
        pltpu.make_async_copy(g_hbm.at[src_v.at[i + 1]], rows1, gs1).wait()
        pltpu.sync_copy(rows1, acc_sh.at[dst_v.at[i + 1]], add=True)
        return carry

      lax.fori_loop(0, nh // 2, epair, 0)
      pltpu.make_async_copy(g_hbm.at[src_v.at[nh - 1]], rows0, gs0).wait()

    plsc.subcore_barrier()
    pltpu.sync_copy(acc_sh.at[pl.ds(sid * _WSTEP, _WROWS)],
                    out_hbm.at[pl.ds(cid * _N + sid * _WSTEP, _WROWS)])

  return segsum


_segsum_hid = _make_segsum(_DHID, _CHUNK2, 2)
_segsum_cls = _make_segsum(_DP, _CHUNK2, 1)

_deg_mesh = plsc.VectorSubcoreMesh(core_axis_name="c", subcore_axis_name="s")


@functools.partial(
    pl.kernel,
    out_type=jax.ShapeDtypeStruct((_NC * _N, 16), jnp.float32),
    mesh=_deg_mesh,
    compiler_params=pltpu.CompilerParams(use_tc_tiling_on_sc=False,
                                         skip_device_barrier=True),
    scratch_types=[
        pltpu.VMEM((_EPW // _CHUNK2, _CHUNK2), jnp.int32),
        pltpu.VMEM((_CHUNK2, 16), jnp.float32),
        pltpu.VMEM_SHARED((_N, 16), jnp.float32),
        pltpu.SemaphoreType.DMA,
    ],
)
def _sc_deg(dst_hbm, ones_hbm, z_hbm, out_hbm, dst_v, ones_v, acc_sh, sem):
  cid = lax.axis_index("c")
  sid = lax.axis_index("s")
  wid = cid * _NS + sid
  pltpu.sync_copy(z_hbm, acc_sh.at[pl.ds(sid * _WSTEP, _WROWS)])
  pltpu.sync_copy(dst_hbm.at[pl.ds(wid * (_EPW // _CHUNK2), _EPW // _CHUNK2)], dst_v)
  pltpu.sync_copy(ones_hbm, ones_v)
  plsc.subcore_barrier()

  def egroup(g, carry):
    def fire(j, c):
      pltpu.async_copy(ones_v, acc_sh.at[dst_v.at[g * 40 + j]], sem, add=True)
      return c

    lax.fori_loop(0, 40, fire, 0)

    def drain(j, c):
      pltpu.make_async_copy(ones_v, acc_sh.at[dst_v.at[0]], sem).wait()
      return c

    lax.fori_loop(0, 40, drain, 0)
    return carry

  lax.fori_loop(0, (_EPW // _CHUNK2) // 40, egroup, 0)
  plsc.subcore_barrier()
  pltpu.sync_copy(acc_sh.at[pl.ds(sid * _WSTEP, _WROWS)],
                  out_hbm.at[pl.ds(cid * _N + sid * _WSTEP, _WROWS)])


def _tc_prep(x, W1, degp):
  def body(x_ref, w_ref, d_ref, g_ref, dinv_ref):
    deg = d_ref[0, :, 0:1] + d_ref[1, :, 0:1] + 1.0
    dinv = lax.rsqrt(deg)
    g_ref[...] = jnp.dot(x_ref[...], w_ref[...],
                         preferred_element_type=jnp.float32) * dinv
    dinv_ref[...] = dinv

  return pl.pallas_call(
      body,
      grid=(_N // _BN,),
      in_specs=[
          pl.BlockSpec((_BN, _DIN), lambda i: (i, 0)),
          pl.BlockSpec((_DIN, _DHID), lambda i: (0, 0)),
          pl.BlockSpec((2, _BN, 16), lambda i: (0, i, 0)),
      ],
      out_specs=[
          pl.BlockSpec((_BN, _DHID), lambda i: (i, 0)),
          pl.BlockSpec((_BN, 1), lambda i: (i, 0)),
      ],
      out_shape=[
          jax.ShapeDtypeStruct((_N, _DHID), jnp.float32),
          jax.ShapeDtypeStruct((_N, 1), jnp.float32),
      ],
  )(x, W1, degp)


def _tc_mid(s1, g1, dinv, b1, W2p):
  def body(s_ref, g_ref, dv_ref, b_ref, w_ref, g2_ref):
    dinv = dv_ref[...]
    o1 = (s_ref[0] + s_ref[1] + g_ref[...]) * dinv + b_ref[...]
    o1 = jnp.maximum(o1, 0.0)
    g2_ref[...] = jnp.dot(o1, w_ref[...],
                          preferred_element_type=jnp.float32) * dinv

  return pl.pallas_call(
      body,
      grid=(_N // _BN,),
      in_specs=[
          pl.BlockSpec((2, _BN, _DHID), lambda i: (0, i, 0)),
          pl.BlockSpec((_BN, _DHID), lambda i: (i, 0)),
          pl.BlockSpec((_BN, 1), lambda i: (i, 0)),
          pl.BlockSpec((1, _DHID), lambda i: (0, 0)),
          pl.BlockSpec((_DHID, _DP), lambda i: (0, 0)),
      ],
      out_specs=pl.BlockSpec((_BN, _DP), lambda i: (i, 0)),
      out_shape=jax.ShapeDtypeStruct((_N, _DP), jnp.float32),
  )(s1, g1, dinv, b1, W2p)


def _tc_final(s2, g2, dinv, b2p):
  def body(s_ref, g_ref, dv_ref, b_ref, o_ref):
    dinv = dv_ref[...]
    t = (s_ref[0] + s_ref[1] + g_ref[...]) * dinv + b_ref[...]
    t = jnp.maximum(t, 0.0)
    mask = lax.broadcasted_iota(jnp.int32, (_BN, _DP), 1) < _NCLS
    t = jnp.where(mask, t, -1e30)
    m = jnp.max(t, axis=1, keepdims=True)
    e = jnp.exp(t - m)
    s = jnp.sum(e, axis=1, keepdims=True)
    o_ref[...] = (t - m - jnp.log(s))[:, :_NCLS]

  return pl.pallas_call(
      body,
      grid=(_N // _BN,),
      in_specs=[
          pl.BlockSpec((2, _BN, _DP), lambda i: (0, i, 0)),
          pl.BlockSpec((_BN, _DP), lambda i: (i, 0)),
          pl.BlockSpec((_BN, 1), lambda i: (i, 0)),
          pl.BlockSpec((1, _DP), lambda i: (0, 0)),
      ],
      out_specs=pl.BlockSpec((_BN, _NCLS), lambda i: (i, 0)),
      out_shape=jax.ShapeDtypeStruct((_N, _NCLS), jnp.float32),
  )(s2, g2, dinv, b2p)


def kernel(x, edge_index, W1, b1, W2, b2):
  src2 = edge_index[0].astype(jnp.int32).reshape(_E // _CHUNK2, _CHUNK2)
  dst2 = edge_index[1].astype(jnp.int32).reshape(_E // _CHUNK2, _CHUNK2)

  ones16 = jnp.ones((_CHUNK2, 16), jnp.float32)
  z16 = jnp.zeros((_WROWS, 16), jnp.float32)

  degp = _sc_deg(dst2, ones16, z16).reshape(2, _N, 16)
  g1, dinv = _tc_prep(x, W1, degp)
  s1 = _segsum_hid(g1, src2, dst2).reshape(2, _N, _DHID)
  g2 = _tc_mid(s1, g1, dinv, b1.reshape(1, _DHID),
               jnp.zeros((_DHID, _DP), jnp.float32).at[:, :_NCLS].set(W2))
  s2 = _segsum_cls(g2, src2, dst2).reshape(2, _N, _DP)
  return _tc_final(s2, g2, dinv,
                   jnp.zeros((1, _DP), jnp.float32).at[0, :_NCLS].set(b2))
